# Initial kernel scaffold; baseline (speedup 1.0000x reference)
import jax
import jax.numpy as jnp
from jax.experimental import pallas as pl


def kernel(v, b, k, q, labels, W_img, b_img, emb, W_q, b_q, W_k, b_k, W_katt1, W_katt2, W_gate, W_cells, b_cells, Wc, bc):
    raise NotImplementedError("write your pallas kernel here")


# speedup vs baseline: 1.4921x; 1.4921x over previous
import functools

import jax
import jax.numpy as jnp
from jax import lax
from jax.experimental import pallas as pl
from jax.experimental.pallas import tpu as pltpu
from jax.experimental.pallas import tpu_sc as plsc

NUM_HID = 512
NUM_CELL = 5
LAYERS = 10
NUM_ANS = 3129
BS = 64
NUM_R = 36
Q_LEN = 20
K_LEN = 50
D_PAD = 304

B_TILES = 8
B_PER_TILE = BS // B_TILES


def _sc_gather(table_pad, idx):
  info = plsc.get_sparse_core_info()
  nc, ns = info.num_cores, info.num_subcores
  nw = nc * ns
  b_total = idx.shape[0]
  b_per_w = b_total // nw
  mesh = plsc.VectorSubcoreMesh(core_axis_name="c", subcore_axis_name="s")

  @functools.partial(
      pl.kernel,
      mesh=mesh,
      out_type=jax.ShapeDtypeStruct((b_total, D_PAD), jnp.float32),
      scratch_types=[
          pltpu.VMEM((b_per_w,), jnp.int32),
          pltpu.VMEM((b_per_w, D_PAD), jnp.float32),
          pltpu.SemaphoreType.DMA,
      ],
      compiler_params=pltpu.CompilerParams(use_tc_tiling_on_sc=False),
  )
  def gather_kernel(table_hbm, idx_hbm, out_hbm, idx_v, rows_v, sem):
    wid = lax.axis_index("s") * nc + lax.axis_index("c")
    base = wid * b_per_w
    pltpu.sync_copy(idx_hbm.at[pl.ds(base, b_per_w)], idx_v)
    pltpu.async_copy(table_hbm.at[idx_v], rows_v, sem).wait()
    pltpu.sync_copy(rows_v, out_hbm.at[pl.ds(base, b_per_w)])

  return gather_kernel(table_pad, idx)


def _ctx_body(wemb_ref, wq_ref, bq_ref, k_ref, wk_ref, bk_ref, wk1_ref,
              wk2_ref, qk_ref, qsum_ref):
  i = pl.program_id(0)
  we = wemb_ref[...].reshape(B_PER_TILE * Q_LEN, D_PAD)
  q_emb = jnp.tanh(
      jnp.dot(we, wq_ref[...], preferred_element_type=jnp.float32)
      + bq_ref[...])
  q_ctx = jnp.mean(q_emb.reshape(B_PER_TILE, Q_LEN, NUM_HID), axis=1)
  kk = k_ref[...].reshape(B_PER_TILE * K_LEN, 1024)
  k_emb = jnp.tanh(
      jnp.dot(kk, wk_ref[...], preferred_element_type=jnp.float32)
      + bk_ref[...])
  k_emb = jnp.maximum(
      jnp.dot(k_emb, wk1_ref[...], preferred_element_type=jnp.float32), 0.0)
  k_emb = jnp.maximum(
      jnp.dot(k_emb, wk2_ref[...], preferred_element_type=jnp.float32), 0.0)
  k_ctx = jnp.mean(k_emb.reshape(B_PER_TILE, K_LEN, NUM_HID), axis=1)
  qk_ref[...] = q_ctx + k_ctx

  @pl.when(i == 0)
  def _():
    qsum_ref[...] = jnp.zeros_like(qsum_ref)

  qsum_ref[...] += jnp.sum(q_emb).reshape(1, 1)


def _ctx_call(wemb, W_q_pad, b_q, k, W_k, b_k, W_katt1, W_katt2):
  return pl.pallas_call(
      _ctx_body,
      grid=(B_TILES,),
      in_specs=[
          pl.BlockSpec((B_PER_TILE, Q_LEN, D_PAD), lambda i: (i, 0, 0)),
          pl.BlockSpec((D_PAD, NUM_HID), lambda i: (0, 0)),
          pl.BlockSpec((1, NUM_HID), lambda i: (0, 0)),
          pl.BlockSpec((B_PER_TILE, K_LEN, 1024), lambda i: (i, 0, 0)),
          pl.BlockSpec((1024, NUM_HID), lambda i: (0, 0)),
          pl.BlockSpec((1, NUM_HID), lambda i: (0, 0)),
          pl.BlockSpec((NUM_HID, NUM_HID), lambda i: (0, 0)),
          pl.BlockSpec((NUM_HID, NUM_HID), lambda i: (0, 0)),
      ],
      out_specs=[
          pl.BlockSpec((B_PER_TILE, NUM_HID), lambda i: (i, 0)),
          pl.BlockSpec((1, 1), lambda i: (0, 0)),
      ],
      out_shape=[
          jax.ShapeDtypeStruct((BS, NUM_HID), jnp.float32),
          jax.ShapeDtypeStruct((1, 1), jnp.float32),
      ],
  )(wemb, W_q_pad, b_q, k, W_k, b_k, W_katt1, W_katt2)


def _moe_body(v_ref, wimg_ref, bimg_ref, qk_ref, wg_ref, wc_ref, bcell_ref,
              vsum_ref):
  i = pl.program_id(0)
  rows = B_PER_TILE * NUM_R
  v2 = v_ref[...].reshape(rows, 4 * NUM_HID)
  v_emb = (jnp.dot(v2, wimg_ref[...], preferred_element_type=jnp.float32)
           + bimg_ref[...])
  ctx_base = jnp.broadcast_to(
      qk_ref[...][:, None, :], (B_PER_TILE, NUM_R, NUM_HID)
  ).reshape(rows, NUM_HID)

  mm = jnp.zeros_like(v_emb)
  last = [v_emb] * NUM_CELL
  for _ in range(LAYERS):
    ctx = mm + (last[0] + last[1] + last[2] + last[3] + last[4]) * (
        1.0 / NUM_CELL) + ctx_base
    z = jnp.dot(ctx, wg_ref[...], preferred_element_type=jnp.float32)
    z = z - jnp.max(z, axis=-1, keepdims=True)
    ez = jnp.exp(z)
    gate = ez / jnp.sum(ez, axis=-1, keepdims=True)
    outs = [
        jnp.dot(last[c], wc_ref[c], preferred_element_type=jnp.float32)
        + bcell_ref[c][None, :]
        for c in range(NUM_CELL)
    ]
    mm = mm + sum(gate[:, c:c + 1] * outs[c] for c in range(NUM_CELL))
    last = [jnp.maximum(o, 0.0) for o in outs]

  @pl.when(i == 0)
  def _():
    vsum_ref[...] = jnp.zeros_like(vsum_ref)

  vsum_ref[...] += jnp.sum(mm).reshape(1, 1)


def _moe_call(v, W_img, b_img, qk_ctx, W_gate, W_cells, b_cells):
  return pl.pallas_call(
      _moe_body,
      grid=(B_TILES,),
      in_specs=[
          pl.BlockSpec((B_PER_TILE, NUM_R, 4 * NUM_HID), lambda i: (i, 0, 0)),
          pl.BlockSpec((4 * NUM_HID, NUM_HID), lambda i: (0, 0)),
          pl.BlockSpec((1, NUM_HID), lambda i: (0, 0)),
          pl.BlockSpec((B_PER_TILE, NUM_HID), lambda i: (i, 0)),
          pl.BlockSpec((NUM_HID, NUM_CELL), lambda i: (0, 0)),
          pl.BlockSpec((NUM_CELL, NUM_HID, NUM_HID), lambda i: (0, 0, 0)),
          pl.BlockSpec((NUM_CELL, NUM_HID), lambda i: (0, 0)),
      ],
      out_specs=pl.BlockSpec((1, 1), lambda i: (0, 0)),
      out_shape=jax.ShapeDtypeStruct((1, 1), jnp.float32),
  )(v, W_img, b_img, qk_ctx, W_gate, W_cells, b_cells)


def _logits_body(wc_ref, bc_ref, qsum_ref, vsum_ref, out_ref):
  s = (qsum_ref[0, 0] * (1.0 / (BS * Q_LEN * NUM_HID))
       + vsum_ref[0, 0] * (1.0 / (BS * NUM_R * NUM_HID)))
  out_ref[...] = s * jnp.sum(wc_ref[...], axis=0, keepdims=True) + bc_ref[...]


def _logits_call(Wc, bc2, qsum, vsum):
  return pl.pallas_call(
      _logits_body,
      out_shape=jax.ShapeDtypeStruct((1, NUM_ANS), jnp.float32),
  )(Wc, bc2, qsum, vsum)


def kernel(v, b, k, q, labels, W_img, b_img, emb, W_q, b_q, W_k, b_k,
           W_katt1, W_katt2, W_gate, W_cells, b_cells, Wc, bc):
  del b, labels
  emb_pad = jnp.pad(emb, ((0, 0), (0, D_PAD - emb.shape[1])))
  idx = q.reshape(-1).astype(jnp.int32)
  wemb_flat = _sc_gather(emb_pad, idx)
  wemb = wemb_flat.reshape(BS, Q_LEN, D_PAD)

  W_q_pad = jnp.pad(W_q, ((0, D_PAD - W_q.shape[0]), (0, 0)))
  qk_ctx, qsum = _ctx_call(wemb, W_q_pad, b_q.reshape(1, -1), k, W_k,
                           b_k.reshape(1, -1), W_katt1, W_katt2)
  vsum = _moe_call(v, W_img, b_img.reshape(1, -1), qk_ctx, W_gate, W_cells,
                   b_cells)
  logits = _logits_call(Wc, bc.reshape(1, -1), qsum, vsum)
  return logits.reshape(NUM_ANS)
